# initial kernel scaffold (unmeasured)
import jax
import jax.numpy as jnp
from jax import lax
from jax.experimental import pallas as pl
from jax.experimental.pallas import tpu as pltpu


def kernel(
    x,
):
    def body(*refs):
        pass

    out_shape = jax.ShapeDtypeStruct(..., jnp.float32)
    return pl.pallas_call(body, out_shape=out_shape)(...)


# baseline (device time: 595534 ns/iter reference)
import jax
import jax.numpy as jnp
from jax import lax
from jax.experimental import pallas as pl
from jax.experimental.pallas import tpu as pltpu

N_Y = 4
N_HOPS = 2 * (N_Y - 1)


def kernel(x):
    m, n = x.shape
    blk = m // N_Y

    def body(x_hbm, out_hbm, comm, xb, load_sem, store_sem,
             send_sems, recv_sems, credit_sems):
        my_x = lax.axis_index("x")
        my_y = lax.axis_index("y")
        my_z = lax.axis_index("z")
        left = lax.rem(my_y + (N_Y - 1), N_Y)
        right = lax.rem(my_y + 1, N_Y)

        def blk_of(k):
            return lax.rem(k, N_Y)

        def load_x_block(g):
            cp = pltpu.make_async_copy(
                x_hbm.at[pl.ds(g * blk, blk), :], xb, load_sem)
            cp.start()
            return cp

        def store_block(slot, g):
            cp = pltpu.make_async_copy(
                comm.at[slot], out_hbm.at[pl.ds(g * blk, blk), :], store_sem)
            cp.start()
            return cp

        ld = load_x_block(blk_of(my_y))

        barrier_sem = pltpu.get_barrier_semaphore()
        for nbr in (left, right):
            pl.semaphore_signal(
                barrier_sem, inc=1,
                device_id=(my_x, nbr, my_z),
                device_id_type=pl.DeviceIdType.MESH,
            )
        pl.semaphore_wait(barrier_sem, 2)

        ld.wait()
        comm[0, :, :] = xb[:, :].astype(jnp.bfloat16)

        own_store = None
        for s in range(N_HOPS):
            send_slot = s % 3
            recv_slot = (s + 1) % 3

            if s >= 3:
                pl.semaphore_wait(credit_sems.at[recv_slot], 1)

            rdma = pltpu.make_async_remote_copy(
                src_ref=comm.at[send_slot],
                dst_ref=comm.at[recv_slot],
                send_sem=send_sems.at[send_slot],
                recv_sem=recv_sems.at[recv_slot],
                device_id=(my_x, right, my_z),
                device_id_type=pl.DeviceIdType.MESH,
            )
            rdma.start()

            if s < N_Y - 1:
                ld = load_x_block(blk_of(my_y + N_Y - 1 - s))

            rdma.wait()

            if s < N_Y - 1:
                ld.wait()
                acc = comm[recv_slot, :, :].astype(jnp.float32) + xb[:, :]
                comm[recv_slot, :, :] = acc.astype(jnp.bfloat16)
                if s == N_Y - 2:
                    own_store = store_block(recv_slot, blk_of(my_y + 1))
            else:
                g = blk_of(my_y + N_Y - (s - (N_Y - 1)))
                st = store_block(recv_slot, g)
                st.wait()

            if 1 <= s <= 3:
                if s == 3 and own_store is not None:
                    own_store.wait()
                pl.semaphore_signal(
                    credit_sems.at[send_slot], inc=1,
                    device_id=(my_x, left, my_z),
                    device_id_type=pl.DeviceIdType.MESH,
                )

    out_shape = jax.ShapeDtypeStruct((m, n), jnp.bfloat16)
    return pl.pallas_call(
        body,
        out_shape=out_shape,
        in_specs=[pl.BlockSpec(memory_space=pl.ANY)],
        out_specs=pl.BlockSpec(memory_space=pl.ANY),
        scratch_shapes=[
            pltpu.VMEM((3, blk, n), jnp.bfloat16),
            pltpu.VMEM((blk, n), jnp.float32),
            pltpu.SemaphoreType.DMA,
            pltpu.SemaphoreType.DMA,
            pltpu.SemaphoreType.DMA((3,)),
            pltpu.SemaphoreType.DMA((3,)),
            pltpu.SemaphoreType.REGULAR((3,)),
        ],
        compiler_params=pltpu.CompilerParams(
            collective_id=0,
            vmem_limit_bytes=56 * 1024 * 1024,
        ),
    )(x)


# device time: 292568 ns/iter; 2.0355x vs baseline; 2.0355x over previous
import jax
import jax.numpy as jnp
from jax import lax
from jax.experimental import pallas as pl
from jax.experimental.pallas import tpu as pltpu

N_Y = 4
N_XZ = 8


def kernel(x):
    m, n = x.shape
    colblk = m // N_XZ
    half = colblk // 2
    sub = colblk // N_Y

    def body(x_hbm, out_hbm, acomm, cw, ccw, xb,
             load_sem, store_sems,
             a_send, a_recv, cw_send, cw_recv, ccw_send, ccw_recv):
        my_x = lax.axis_index("x")
        my_y = lax.axis_index("y")
        my_z = lax.axis_index("z")
        yl = lax.rem(my_y + N_Y - 1, N_Y)
        yr = lax.rem(my_y + 1, N_Y)

        p = jnp.where(my_x == 0, my_z, 7 - my_z)

        def ring_coords(q):
            q = lax.rem(q, N_XZ)
            return jnp.where(q < 4, 0, 1), jnp.where(q < 4, q, 7 - q)

        nx_x, nx_z = ring_coords(p + 1)
        pv_x, pv_z = ring_coords(p + N_XZ - 1)
        row0 = p * colblk

        def load_sub(sb):
            cp = pltpu.make_async_copy(
                x_hbm.at[pl.ds(row0 + sb * sub, sub), :], xb, load_sem)
            cp.start()
            return cp

        ld = load_sub(my_y)

        barrier_sem = pltpu.get_barrier_semaphore()
        for dev in ((my_x, yl, my_z), (my_x, yr, my_z),
                    (nx_x, my_y, nx_z), (pv_x, my_y, pv_z)):
            pl.semaphore_signal(
                barrier_sem, inc=1, device_id=dev,
                device_id_type=pl.DeviceIdType.MESH)
        pl.semaphore_wait(barrier_sem, 4)

        ld.wait()
        acomm[0, :, :] = xb[:, :].astype(jnp.bfloat16)

        for h in range(2 * (N_Y - 1)):
            rdma = pltpu.make_async_remote_copy(
                src_ref=acomm.at[h],
                dst_ref=acomm.at[h + 1],
                send_sem=a_send.at[h],
                recv_sem=a_recv.at[h],
                device_id=(my_x, yr, my_z),
                device_id_type=pl.DeviceIdType.MESH,
            )
            rdma.start()
            if h < N_Y - 1:
                ld = load_sub(lax.rem(my_y + N_Y - 1 - h, N_Y))
            rdma.wait()
            if h < N_Y - 1:
                ld.wait()
                acc = acomm[h + 1, :, :].astype(jnp.float32) + xb[:, :]
                acomm[h + 1, :, :] = acc.astype(jnp.bfloat16)

        for j in range(N_Y - 1, 2 * N_Y - 1):
            sbj = lax.rem(my_y + 2 * N_Y - j, N_Y)
            off = lax.rem(sbj, 2) * sub
            in_cw = sbj < 2

            @pl.when(in_cw)
            def _():
                cw[0, pl.ds(off, sub), :] = acomm[j, :, :]

            @pl.when(jnp.logical_not(in_cw))
            def _():
                ccw[0, pl.ds(off, sub), :] = acomm[j, :, :]

        st_cw = pltpu.make_async_copy(
            cw.at[0], out_hbm.at[pl.ds(row0, half), :], store_sems.at[0])
        st_ccw = pltpu.make_async_copy(
            ccw.at[0], out_hbm.at[pl.ds(row0 + half, half), :],
            store_sems.at[1])
        st_cw.start()
        st_ccw.start()
        st_cw.wait()
        st_ccw.wait()

        import os
        _NB = int(os.environ.get("KERNEL_B_HOPS", str(N_XZ - 1)))
        _DIRS = os.environ.get("KERNEL_B_DIRS", "both")
        for k in range(_NB):
            r_cw = pltpu.make_async_remote_copy(
                src_ref=cw.at[k], dst_ref=cw.at[k + 1],
                send_sem=cw_send.at[k], recv_sem=cw_recv.at[k],
                device_id=(nx_x, my_y, nx_z),
                device_id_type=pl.DeviceIdType.MESH,
            )
            r_ccw = pltpu.make_async_remote_copy(
                src_ref=ccw.at[k], dst_ref=ccw.at[k + 1],
                send_sem=ccw_send.at[k], recv_sem=ccw_recv.at[k],
                device_id=(pv_x, my_y, pv_z),
                device_id_type=pl.DeviceIdType.MESH,
            )
            if _DIRS in ("both", "cw"):
                r_cw.start()
            if _DIRS in ("both", "ccw"):
                r_ccw.start()
            if _DIRS in ("both", "cw"):
                r_cw.wait()
            if _DIRS in ("both", "ccw"):
                r_ccw.wait()

            g_cw = lax.rem(p + N_XZ - k - 1, N_XZ)
            g_ccw = lax.rem(p + k + 1, N_XZ)
            s1 = pltpu.make_async_copy(
                cw.at[k + 1],
                out_hbm.at[pl.ds(g_cw * colblk, half), :],
                store_sems.at[0])
            s2 = pltpu.make_async_copy(
                ccw.at[k + 1],
                out_hbm.at[pl.ds(g_ccw * colblk + half, half), :],
                store_sems.at[1])
            s1.start()
            s2.start()
            s1.wait()
            s2.wait()

    out_shape = jax.ShapeDtypeStruct((m, n), jnp.bfloat16)
    return pl.pallas_call(
        body,
        out_shape=out_shape,
        in_specs=[pl.BlockSpec(memory_space=pl.ANY)],
        out_specs=pl.BlockSpec(memory_space=pl.ANY),
        scratch_shapes=[
            pltpu.VMEM((2 * N_Y - 1, sub, n), jnp.bfloat16),
            pltpu.VMEM((N_XZ, half, n), jnp.bfloat16),
            pltpu.VMEM((N_XZ, half, n), jnp.bfloat16),
            pltpu.VMEM((sub, n), jnp.float32),
            pltpu.SemaphoreType.DMA,
            pltpu.SemaphoreType.DMA((2,)),
            pltpu.SemaphoreType.DMA((2 * N_Y - 2,)),
            pltpu.SemaphoreType.DMA((2 * N_Y - 2,)),
            pltpu.SemaphoreType.DMA((N_XZ - 1,)),
            pltpu.SemaphoreType.DMA((N_XZ - 1,)),
            pltpu.SemaphoreType.DMA((N_XZ - 1,)),
            pltpu.SemaphoreType.DMA((N_XZ - 1,)),
        ],
        compiler_params=pltpu.CompilerParams(
            collective_id=0,
            vmem_limit_bytes=56 * 1024 * 1024,
        ),
    )(x)


# device time: 278942 ns/iter; 2.1350x vs baseline; 1.0488x over previous
import jax
import jax.numpy as jnp
from jax import lax
from jax.experimental import pallas as pl
from jax.experimental.pallas import tpu as pltpu

N_Y = 4
N_XZ = 8


def kernel(x):
    m, n = x.shape
    colblk = m // N_XZ
    half = colblk // 2
    sub = colblk // N_Y

    def body(x_hbm, out_hbm, acomm, cw, ccw, xb,
             load_sem, store_sems,
             a_send, a_recv, cw_send, cw_recv, ccw_send, ccw_recv):
        my_x = lax.axis_index("x")
        my_y = lax.axis_index("y")
        my_z = lax.axis_index("z")
        yl = lax.rem(my_y + N_Y - 1, N_Y)
        yr = lax.rem(my_y + 1, N_Y)

        p = jnp.where(my_x == 0, my_z, 7 - my_z)

        def ring_coords(q):
            q = lax.rem(q, N_XZ)
            return jnp.where(q < 4, 0, 1), jnp.where(q < 4, q, 7 - q)

        nx_x, nx_z = ring_coords(p + 1)
        pv_x, pv_z = ring_coords(p + N_XZ - 1)
        row0 = p * colblk

        def load_sub(sb):
            cp = pltpu.make_async_copy(
                x_hbm.at[pl.ds(row0 + sb * sub, sub), :], xb, load_sem)
            cp.start()
            return cp

        ld = load_sub(my_y)

        barrier_sem = pltpu.get_barrier_semaphore()
        for dev in ((my_x, yl, my_z), (my_x, yr, my_z),
                    (nx_x, my_y, nx_z), (pv_x, my_y, pv_z)):
            pl.semaphore_signal(
                barrier_sem, inc=1, device_id=dev,
                device_id_type=pl.DeviceIdType.MESH)
        pl.semaphore_wait(barrier_sem, 4)

        ld.wait()
        acomm[0, :, :] = xb[:, :].astype(jnp.bfloat16)

        a_rdmas = []
        for h in range(2 * (N_Y - 1)):
            rdma = pltpu.make_async_remote_copy(
                src_ref=acomm.at[h],
                dst_ref=acomm.at[h + 1],
                send_sem=a_send.at[h],
                recv_sem=a_recv.at[h],
                device_id=(my_x, yr, my_z),
                device_id_type=pl.DeviceIdType.MESH,
            )
            rdma.start()
            a_rdmas.append(rdma)
            if h < N_Y - 1:
                ld = load_sub(lax.rem(my_y + N_Y - 1 - h, N_Y))
            rdma.wait_recv()
            if h < N_Y - 1:
                ld.wait()
                acc = acomm[h + 1, :, :].astype(jnp.float32) + xb[:, :]
                acomm[h + 1, :, :] = acc.astype(jnp.bfloat16)

        for j in range(N_Y - 1, 2 * N_Y - 1):
            sbj = lax.rem(my_y + 2 * N_Y - j, N_Y)
            off = lax.rem(sbj, 2) * sub
            in_cw = sbj < 2

            @pl.when(in_cw)
            def _():
                cw[0, pl.ds(off, sub), :] = acomm[j, :, :]

            @pl.when(jnp.logical_not(in_cw))
            def _():
                ccw[0, pl.ds(off, sub), :] = acomm[j, :, :]

        stores = []
        st_cw = pltpu.make_async_copy(
            cw.at[0], out_hbm.at[pl.ds(row0, half), :], store_sems.at[0])
        st_ccw = pltpu.make_async_copy(
            ccw.at[0], out_hbm.at[pl.ds(row0 + half, half), :],
            store_sems.at[1])
        st_cw.start()
        st_ccw.start()
        stores += [st_cw, st_ccw]

        b_rdmas = []

        def b_hop(k):
            r_cw = pltpu.make_async_remote_copy(
                src_ref=cw.at[k], dst_ref=cw.at[k + 1],
                send_sem=cw_send.at[k], recv_sem=cw_recv.at[k],
                device_id=(nx_x, my_y, nx_z),
                device_id_type=pl.DeviceIdType.MESH,
            )
            r_ccw = pltpu.make_async_remote_copy(
                src_ref=ccw.at[k], dst_ref=ccw.at[k + 1],
                send_sem=ccw_send.at[k], recv_sem=ccw_recv.at[k],
                device_id=(pv_x, my_y, pv_z),
                device_id_type=pl.DeviceIdType.MESH,
            )
            r_cw.start()
            r_ccw.start()
            b_rdmas.append((r_cw, r_ccw))
            return r_cw, r_ccw

        r_cw, r_ccw = b_hop(0)
        for k in range(N_XZ - 1):
            r_cw.wait_recv()
            r_ccw.wait_recv()
            if k + 1 < N_XZ - 1:
                r_cw, r_ccw = b_hop(k + 1)

            g_cw = lax.rem(p + N_XZ - k - 1, N_XZ)
            g_ccw = lax.rem(p + k + 1, N_XZ)
            s1 = pltpu.make_async_copy(
                cw.at[k + 1],
                out_hbm.at[pl.ds(g_cw * colblk, half), :],
                store_sems.at[2 + 2 * k])
            s2 = pltpu.make_async_copy(
                ccw.at[k + 1],
                out_hbm.at[pl.ds(g_ccw * colblk + half, half), :],
                store_sems.at[3 + 2 * k])
            s1.start()
            s2.start()
            stores += [s1, s2]

        for rdma in a_rdmas:
            rdma.wait_send()
        for r_cw, r_ccw in b_rdmas:
            r_cw.wait_send()
            r_ccw.wait_send()
        for st in stores:
            st.wait()

    out_shape = jax.ShapeDtypeStruct((m, n), jnp.bfloat16)
    return pl.pallas_call(
        body,
        out_shape=out_shape,
        in_specs=[pl.BlockSpec(memory_space=pl.ANY)],
        out_specs=pl.BlockSpec(memory_space=pl.ANY),
        scratch_shapes=[
            pltpu.VMEM((2 * N_Y - 1, sub, n), jnp.bfloat16),
            pltpu.VMEM((N_XZ, half, n), jnp.bfloat16),
            pltpu.VMEM((N_XZ, half, n), jnp.bfloat16),
            pltpu.VMEM((sub, n), jnp.float32),
            pltpu.SemaphoreType.DMA,
            pltpu.SemaphoreType.DMA((2 * N_XZ,)),
            pltpu.SemaphoreType.DMA((2 * N_Y - 2,)),
            pltpu.SemaphoreType.DMA((2 * N_Y - 2,)),
            pltpu.SemaphoreType.DMA((N_XZ - 1,)),
            pltpu.SemaphoreType.DMA((N_XZ - 1,)),
            pltpu.SemaphoreType.DMA((N_XZ - 1,)),
            pltpu.SemaphoreType.DMA((N_XZ - 1,)),
        ],
        compiler_params=pltpu.CompilerParams(
            collective_id=0,
            vmem_limit_bytes=56 * 1024 * 1024,
        ),
    )(x)


# device time: 254552 ns/iter; 2.3395x vs baseline; 1.0958x over previous
import jax
import jax.numpy as jnp
from jax import lax
from jax.experimental import pallas as pl
from jax.experimental.pallas import tpu as pltpu

N_Y = 4
N_XZ = 8
N_C = 2


def kernel(x):
    m, n = x.shape
    colblk = m // N_XZ
    chunk = colblk // N_C
    bhalf = chunk // 2
    sub = chunk // N_Y

    def body(x_hbm, out_hbm, acomm, cw, ccw, xb,
             load_sems, store_sems,
             a_send, a_recv, cw_send, cw_recv, ccw_send, ccw_recv):
        my_x = lax.axis_index("x")
        my_y = lax.axis_index("y")
        my_z = lax.axis_index("z")
        yr = lax.rem(my_y + 1, N_Y)
        yl = lax.rem(my_y + N_Y - 1, N_Y)

        p = jnp.where(my_x == 0, my_z, 7 - my_z)

        def ring_coords(q):
            q = lax.rem(q, N_XZ)
            return jnp.where(q < 4, 0, 1), jnp.where(q < 4, q, 7 - q)

        nx_x, nx_z = ring_coords(p + 1)
        pv_x, pv_z = ring_coords(p + N_XZ - 1)
        row0 = p * colblk

        def load_sub(c, sb, buf):
            cp = pltpu.make_async_copy(
                x_hbm.at[pl.ds(row0 + c * chunk + sb * sub, sub), :],
                xb.at[buf], load_sems.at[buf])
            cp.start()
            return cp, buf

        ld = load_sub(0, my_y, 0)

        barrier_sem = pltpu.get_barrier_semaphore()
        for dev in ((my_x, yl, my_z), (my_x, yr, my_z),
                    (nx_x, my_y, nx_z), (pv_x, my_y, pv_z)):
            pl.semaphore_signal(
                barrier_sem, inc=1, device_id=dev,
                device_id_type=pl.DeviceIdType.MESH)
        pl.semaphore_wait(barrier_sem, 4)

        a_rdmas = []
        b_rdmas = []
        stores = []

        def a_hop(c, h, ld):
            rdma = pltpu.make_async_remote_copy(
                src_ref=acomm.at[c, h],
                dst_ref=acomm.at[c, h + 1],
                send_sem=a_send.at[c, h],
                recv_sem=a_recv.at[c, h],
                device_id=(my_x, yr, my_z),
                device_id_type=pl.DeviceIdType.MESH,
            )
            rdma.start()
            a_rdmas.append(rdma)
            nld = None
            if h + 1 < N_Y - 1:
                nld = load_sub(c, lax.rem(my_y + N_Y - 2 - h, N_Y),
                               h % 2)
            rdma.wait_recv()
            if h < N_Y - 1:
                cp, buf = ld
                cp.wait()
                acc = (acomm[c, h + 1, :, :].astype(jnp.float32)
                       + xb[buf, :, :])
                acomm[c, h + 1, :, :] = acc.astype(jnp.bfloat16)
            return nld

        def scatter(c):
            for j in range(N_Y - 1, 2 * N_Y - 1):
                sbj = lax.rem(my_y + 2 * N_Y - j, N_Y)
                off = lax.rem(sbj, 2) * sub
                in_cw = sbj < 2

                @pl.when(in_cw)
                def _():
                    cw[c, 0, pl.ds(off, sub), :] = acomm[c, j, :, :]

                @pl.when(jnp.logical_not(in_cw))
                def _():
                    ccw[c, 0, pl.ds(off, sub), :] = acomm[c, j, :, :]

            st1 = pltpu.make_async_copy(
                cw.at[c, 0],
                out_hbm.at[pl.ds(row0 + c * chunk, bhalf), :],
                store_sems.at[c, 0])
            st2 = pltpu.make_async_copy(
                ccw.at[c, 0],
                out_hbm.at[pl.ds(row0 + c * chunk + bhalf, bhalf), :],
                store_sems.at[c, 1])
            st1.start()
            st2.start()
            stores.extend([st1, st2])

        def b_start(c, k):
            r_cw = pltpu.make_async_remote_copy(
                src_ref=cw.at[c, k], dst_ref=cw.at[c, k + 1],
                send_sem=cw_send.at[c, k], recv_sem=cw_recv.at[c, k],
                device_id=(nx_x, my_y, nx_z),
                device_id_type=pl.DeviceIdType.MESH,
            )
            r_ccw = pltpu.make_async_remote_copy(
                src_ref=ccw.at[c, k], dst_ref=ccw.at[c, k + 1],
                send_sem=ccw_send.at[c, k], recv_sem=ccw_recv.at[c, k],
                device_id=(pv_x, my_y, pv_z),
                device_id_type=pl.DeviceIdType.MESH,
            )
            r_cw.start()
            r_ccw.start()
            b_rdmas.append((r_cw, r_ccw))
            return r_cw, r_ccw

        def b_service(c, k, pair):
            r_cw, r_ccw = pair
            r_cw.wait_recv()
            r_ccw.wait_recv()
            nxt = b_start(c, k + 1) if k + 1 < N_XZ - 1 else None
            g_cw = lax.rem(p + N_XZ - k - 1, N_XZ)
            g_ccw = lax.rem(p + k + 1, N_XZ)
            s1 = pltpu.make_async_copy(
                cw.at[c, k + 1],
                out_hbm.at[pl.ds(g_cw * colblk + c * chunk, bhalf), :],
                store_sems.at[c, 2 + 2 * k])
            s2 = pltpu.make_async_copy(
                ccw.at[c, k + 1],
                out_hbm.at[
                    pl.ds(g_ccw * colblk + c * chunk + bhalf, bhalf), :],
                store_sems.at[c, 3 + 2 * k])
            s1.start()
            s2.start()
            stores.extend([s1, s2])
            return nxt

        cp, buf = ld
        cp.wait()
        acomm[0, 0, :, :] = xb[buf, :, :].astype(jnp.bfloat16)
        ld = load_sub(0, lax.rem(my_y + N_Y - 1, N_Y), 1)
        for h in range(2 * (N_Y - 1)):
            ld = a_hop(0, h, ld)

        scatter(0)
        b0 = b_start(0, 0)

        cp, buf = load_sub(1, my_y, 0)
        cp.wait()
        acomm[1, 0, :, :] = xb[buf, :, :].astype(jnp.bfloat16)
        ld = load_sub(1, lax.rem(my_y + N_Y - 1, N_Y), 1)
        for k in range(2 * (N_Y - 1)):
            ld = a_hop(1, k, ld)
            b0 = b_service(0, k, b0)

        scatter(1)
        b1 = b_start(1, 0)
        b0 = b_service(0, 2 * (N_Y - 1), b0)

        for k in range(N_XZ - 1):
            b1 = b_service(1, k, b1)

        for rdma in a_rdmas:
            rdma.wait_send()
        for r_cw, r_ccw in b_rdmas:
            r_cw.wait_send()
            r_ccw.wait_send()
        for st in stores:
            st.wait()

    out_shape = jax.ShapeDtypeStruct((m, n), jnp.bfloat16)
    return pl.pallas_call(
        body,
        out_shape=out_shape,
        in_specs=[pl.BlockSpec(memory_space=pl.ANY)],
        out_specs=pl.BlockSpec(memory_space=pl.ANY),
        scratch_shapes=[
            pltpu.VMEM((N_C, 2 * N_Y - 1, sub, n), jnp.bfloat16),
            pltpu.VMEM((N_C, N_XZ, bhalf, n), jnp.bfloat16),
            pltpu.VMEM((N_C, N_XZ, bhalf, n), jnp.bfloat16),
            pltpu.VMEM((2, sub, n), jnp.float32),
            pltpu.SemaphoreType.DMA((2,)),
            pltpu.SemaphoreType.DMA((N_C, 2 * N_XZ)),
            pltpu.SemaphoreType.DMA((N_C, 2 * N_Y - 2)),
            pltpu.SemaphoreType.DMA((N_C, 2 * N_Y - 2)),
            pltpu.SemaphoreType.DMA((N_C, N_XZ - 1)),
            pltpu.SemaphoreType.DMA((N_C, N_XZ - 1)),
            pltpu.SemaphoreType.DMA((N_C, N_XZ - 1)),
            pltpu.SemaphoreType.DMA((N_C, N_XZ - 1)),
        ],
        compiler_params=pltpu.CompilerParams(
            collective_id=0,
            vmem_limit_bytes=56 * 1024 * 1024,
        ),
    )(x)
